# Initial kernel scaffold; baseline (speedup 1.0000x reference)
import functools

import jax
import jax.numpy as jnp
from jax.experimental import pallas as pl
from jax.experimental.pallas import tpu as pltpu

NUM_GROUP = 512
GROUP_SIZE = 32


def _fps_kernel(x_ref, y_ref, z_ref, idx_ref, cx_ref, cy_ref, cz_ref):
    x = x_ref[...]
    y = y_ref[...]
    z = z_ref[...]
    B, N = x.shape
    G = NUM_GROUP
    lane = jax.lax.broadcasted_iota(jnp.int32, (B, N), 1)
    slot = jax.lax.broadcasted_iota(jnp.int32, (B, G), 1)

    def body(t, carry):
        dists, idx, idx_acc, cx_acc, cy_acc, cz_acc = carry
        onehot = lane == idx
        cx = jnp.sum(jnp.where(onehot, x, 0.0), axis=1, keepdims=True)
        cy = jnp.sum(jnp.where(onehot, y, 0.0), axis=1, keepdims=True)
        cz = jnp.sum(jnp.where(onehot, z, 0.0), axis=1, keepdims=True)
        rec = slot == t
        idx_acc = idx_acc + jnp.where(rec, idx, 0)
        cx_acc = cx_acc + jnp.where(rec, cx, 0.0)
        cy_acc = cy_acc + jnp.where(rec, cy, 0.0)
        cz_acc = cz_acc + jnp.where(rec, cz, 0.0)
        d = (x - cx) ** 2 + (y - cy) ** 2 + (z - cz) ** 2
        dists = jnp.minimum(dists, d)
        m = jnp.max(dists, axis=1, keepdims=True)
        nidx = jnp.min(jnp.where(dists == m, lane, N), axis=1, keepdims=True)
        return dists, nidx, idx_acc, cx_acc, cy_acc, cz_acc

    init = (
        jnp.full((B, N), jnp.inf, dtype=jnp.float32),
        jnp.zeros((B, 1), dtype=jnp.int32),
        jnp.zeros((B, G), dtype=jnp.int32),
        jnp.zeros((B, G), dtype=jnp.float32),
        jnp.zeros((B, G), dtype=jnp.float32),
        jnp.zeros((B, G), dtype=jnp.float32),
    )
    _, _, idx_acc, cx_acc, cy_acc, cz_acc = jax.lax.fori_loop(0, G, body, init)
    idx_ref[...] = idx_acc
    cx_ref[...] = cx_acc
    cy_ref[...] = cy_acc
    cz_ref[...] = cz_acc


def _fps(x, y, z):
    B, N = x.shape
    G = NUM_GROUP
    return pl.pallas_call(
        _fps_kernel,
        out_shape=(
            jax.ShapeDtypeStruct((B, G), jnp.int32),
            jax.ShapeDtypeStruct((B, G), jnp.float32),
            jax.ShapeDtypeStruct((B, G), jnp.float32),
            jax.ShapeDtypeStruct((B, G), jnp.float32),
        ),
    )(x, y, z)


def kernel(xyz):
    B, N, _ = xyz.shape
    x = xyz[:, :, 0]
    y = xyz[:, :, 1]
    z = xyz[:, :, 2]
    fps_idx, cx, cy, cz = _fps(x, y, z)
    center = jnp.stack([cx, cy, cz], axis=-1)
    d = jnp.sum((center[:, :, None, :] - xyz[:, None, :, :]) ** 2, axis=-1)
    _, idx = jax.lax.top_k(-d, GROUP_SIZE)
    neighborhood = jax.vmap(lambda pts, i: pts[i])(xyz, idx)
    neighborhood = neighborhood - center[:, :, None, :]
    return (neighborhood, center)


# speedup vs baseline: 1.6961x; 1.6961x over previous
import functools

import jax
import jax.numpy as jnp
from jax.experimental import pallas as pl
from jax.experimental.pallas import tpu as pltpu

NUM_GROUP = 512
GROUP_SIZE = 32


def _fps_kernel(x_ref, y_ref, z_ref, idx_ref, cx_ref, cy_ref, cz_ref):
    x = x_ref[...]
    y = y_ref[...]
    z = z_ref[...]
    B, N = x.shape
    G = NUM_GROUP
    lane = jax.lax.broadcasted_iota(jnp.int32, (B, N), 1)
    slot = jax.lax.broadcasted_iota(jnp.int32, (B, G), 1)

    idx_ref[...] = jnp.zeros((B, G), dtype=jnp.int32)
    cx_ref[...] = jnp.zeros((B, G), dtype=jnp.float32)
    cy_ref[...] = jnp.zeros((B, G), dtype=jnp.float32)
    cz_ref[...] = jnp.zeros((B, G), dtype=jnp.float32)

    def body(t, carry):
        dists, idx = carry
        onehot = lane == idx
        cx = jnp.sum(jnp.where(onehot, x, 0.0), axis=1, keepdims=True)
        cy = jnp.sum(jnp.where(onehot, y, 0.0), axis=1, keepdims=True)
        cz = jnp.sum(jnp.where(onehot, z, 0.0), axis=1, keepdims=True)
        rec = slot == t
        recf = jnp.where(rec, 1.0, 0.0)
        idx_ref[...] = idx_ref[...] + jnp.where(rec, 1, 0) * idx
        cx_ref[...] = cx_ref[...] + recf * cx
        cy_ref[...] = cy_ref[...] + recf * cy
        cz_ref[...] = cz_ref[...] + recf * cz
        d = (x - cx) ** 2 + (y - cy) ** 2 + (z - cz) ** 2
        dists = jnp.minimum(dists, d)
        m = jnp.max(dists, axis=1, keepdims=True)
        nidx = jnp.min(jnp.where(dists == m, lane, N), axis=1, keepdims=True)
        return dists, nidx

    init = (
        jnp.full((B, N), jnp.inf, dtype=jnp.float32),
        jnp.zeros((B, 1), dtype=jnp.int32),
    )
    jax.lax.fori_loop(0, G, body, init)


def _fps(x, y, z):
    B, N = x.shape
    G = NUM_GROUP
    return pl.pallas_call(
        _fps_kernel,
        out_shape=(
            jax.ShapeDtypeStruct((B, G), jnp.int32),
            jax.ShapeDtypeStruct((B, G), jnp.float32),
            jax.ShapeDtypeStruct((B, G), jnp.float32),
            jax.ShapeDtypeStruct((B, G), jnp.float32),
        ),
    )(x, y, z)


def kernel(xyz):
    B, N, _ = xyz.shape
    x = xyz[:, :, 0]
    y = xyz[:, :, 1]
    z = xyz[:, :, 2]
    fps_idx, cx, cy, cz = _fps(x, y, z)
    center = jnp.stack([cx, cy, cz], axis=-1)
    d = jnp.sum((center[:, :, None, :] - xyz[:, None, :, :]) ** 2, axis=-1)
    _, idx = jax.lax.top_k(-d, GROUP_SIZE)
    neighborhood = jax.vmap(lambda pts, i: pts[i])(xyz, idx)
    neighborhood = neighborhood - center[:, :, None, :]
    return (neighborhood, center)


# speedup vs baseline: 2.1240x; 1.2523x over previous
import functools

import jax
import jax.numpy as jnp
from jax import lax
from jax.experimental import pallas as pl
from jax.experimental.pallas import tpu as pltpu
from jax.experimental.pallas import tpu_sc as plsc

NUM_GROUP = 512
GROUP_SIZE = 32
B = 8
N = 8192


def _fps_kernel(x_ref, y_ref, z_ref, idx_ref, cx_ref, cy_ref, cz_ref):
    x = x_ref[...]
    y = y_ref[...]
    z = z_ref[...]
    G = NUM_GROUP
    lane = jax.lax.broadcasted_iota(jnp.int32, (B, N), 1)
    slot = jax.lax.broadcasted_iota(jnp.int32, (B, G), 1)

    idx_ref[...] = jnp.zeros((B, G), dtype=jnp.int32)
    cx_ref[...] = jnp.zeros((B, G), dtype=jnp.float32)
    cy_ref[...] = jnp.zeros((B, G), dtype=jnp.float32)
    cz_ref[...] = jnp.zeros((B, G), dtype=jnp.float32)

    def body(t, carry):
        dists, idx = carry
        onehot = lane == idx
        cx = jnp.sum(jnp.where(onehot, x, 0.0), axis=1, keepdims=True)
        cy = jnp.sum(jnp.where(onehot, y, 0.0), axis=1, keepdims=True)
        cz = jnp.sum(jnp.where(onehot, z, 0.0), axis=1, keepdims=True)
        rec = slot == t
        recf = jnp.where(rec, 1.0, 0.0)
        idx_ref[...] = idx_ref[...] + jnp.where(rec, 1, 0) * idx
        cx_ref[...] = cx_ref[...] + recf * cx
        cy_ref[...] = cy_ref[...] + recf * cy
        cz_ref[...] = cz_ref[...] + recf * cz
        d = (x - cx) ** 2 + (y - cy) ** 2 + (z - cz) ** 2
        dists = jnp.minimum(dists, d)
        m = jnp.max(dists, axis=1, keepdims=True)
        nidx = jnp.min(jnp.where(dists == m, lane, N), axis=1, keepdims=True)
        return dists, nidx

    init = (
        jnp.full((B, N), jnp.inf, dtype=jnp.float32),
        jnp.zeros((B, 1), dtype=jnp.int32),
    )
    jax.lax.fori_loop(0, NUM_GROUP, body, init)


def _fps(x, y, z):
    G = NUM_GROUP
    return pl.pallas_call(
        _fps_kernel,
        out_shape=(
            jax.ShapeDtypeStruct((B, G), jnp.int32),
            jax.ShapeDtypeStruct((B, G), jnp.float32),
            jax.ShapeDtypeStruct((B, G), jnp.float32),
            jax.ShapeDtypeStruct((B, G), jnp.float32),
        ),
    )(x, y, z)


_KW = GROUP_SIZE
_RW = (B * NUM_GROUP) // 32


def _sc_gather_body(
    xs_hbm, ys_hbm, zs_hbm, idx_hbm, cx_hbm, cy_hbm, cz_hbm,
    ox_hbm, oy_hbm, oz_hbm,
    idx_v, gx_v, gy_v, gz_v, cx_v, cy_v, cz_v, ox_v, oy_v, oz_v, sem,
):
    wid = lax.axis_index("s") * 2 + lax.axis_index("c")
    pltpu.sync_copy(idx_hbm.at[wid], idx_v)
    pltpu.sync_copy(cx_hbm.at[wid], cx_v)
    pltpu.sync_copy(cy_hbm.at[wid], cy_v)
    pltpu.sync_copy(cz_hbm.at[wid], cz_v)
    handles = []
    for k in range(_KW):
        handles.append(pltpu.async_copy(xs_hbm.at[idx_v.at[k]], gx_v.at[k], sem))
        handles.append(pltpu.async_copy(ys_hbm.at[idx_v.at[k]], gy_v.at[k], sem))
        handles.append(pltpu.async_copy(zs_hbm.at[idx_v.at[k]], gz_v.at[k], sem))
    for h in handles:
        h.wait()

    for k in range(_KW):
        def rbody(rr, c, k=k):
            s = pl.ds(rr * 16, 16)
            ox_v[k, s] = gx_v[k, s] - cx_v[s]
            oy_v[k, s] = gy_v[k, s] - cy_v[s]
            oz_v[k, s] = gz_v[k, s] - cz_v[s]
            return c

        lax.fori_loop(0, _RW // 16, rbody, 0)
    pltpu.sync_copy(ox_v, ox_hbm.at[wid])
    pltpu.sync_copy(oy_v, oy_hbm.at[wid])
    pltpu.sync_copy(oz_v, oz_hbm.at[wid])


def _sc_gather(xs, ys, zs, idx_g, cx_t, cy_t, cz_t):
    mesh = plsc.VectorSubcoreMesh(core_axis_name="c", subcore_axis_name="s")
    o = jax.ShapeDtypeStruct((32, _KW, _RW), jnp.float32)
    fn = pl.kernel(
        _sc_gather_body,
        mesh=mesh,
        out_type=(o, o, o),
        scratch_types=[
            pltpu.VMEM((_KW, _RW), jnp.int32),
            pltpu.VMEM((_KW, _RW), jnp.float32),
            pltpu.VMEM((_KW, _RW), jnp.float32),
            pltpu.VMEM((_KW, _RW), jnp.float32),
            pltpu.VMEM((_RW,), jnp.float32),
            pltpu.VMEM((_RW,), jnp.float32),
            pltpu.VMEM((_RW,), jnp.float32),
            pltpu.VMEM((_KW, _RW), jnp.float32),
            pltpu.VMEM((_KW, _RW), jnp.float32),
            pltpu.VMEM((_KW, _RW), jnp.float32),
            pltpu.SemaphoreType.DMA,
        ],
    )
    return fn(xs, ys, zs, idx_g, cx_t, cy_t, cz_t)


def _gather_sub(x, y, z, cx, cy, cz, idx):
    idx_g = idx.astype(jnp.int32) + (jnp.arange(B, dtype=jnp.int32) * N)[:, None, None]
    idx_t = (
        idx_g.reshape(B, 4, _RW, _KW).transpose(0, 1, 3, 2).reshape(32, _KW, _RW)
    )
    ox, oy, oz = _sc_gather(
        x.reshape(-1), y.reshape(-1), z.reshape(-1),
        idx_t,
        cx.reshape(32, _RW), cy.reshape(32, _RW), cz.reshape(32, _RW),
    )
    nb = jnp.stack([ox, oy, oz], axis=-1)
    nb = nb.reshape(B, 4, _KW, _RW, 3).transpose(0, 1, 3, 2, 4)
    return nb.reshape(B, NUM_GROUP, GROUP_SIZE, 3)


def kernel(xyz):
    x = xyz[:, :, 0]
    y = xyz[:, :, 1]
    z = xyz[:, :, 2]
    fps_idx, cx, cy, cz = _fps(x, y, z)
    center = jnp.stack([cx, cy, cz], axis=-1)
    d = jnp.sum((center[:, :, None, :] - xyz[:, None, :, :]) ** 2, axis=-1)
    _, idx = jax.lax.top_k(-d, GROUP_SIZE)
    neighborhood = _gather_sub(x, y, z, cx, cy, cz, idx)
    return (neighborhood, center)


# speedup vs baseline: 26.2715x; 12.3691x over previous
import functools

import jax
import jax.numpy as jnp
from jax import lax
from jax.experimental import pallas as pl
from jax.experimental.pallas import tpu as pltpu
from jax.experimental.pallas import tpu_sc as plsc

NUM_GROUP = 512
GROUP_SIZE = 32
B = 8
N = 8192


def _fps_kernel(x_ref, y_ref, z_ref, idx_ref, cx_ref, cy_ref, cz_ref):
    x = x_ref[...]
    y = y_ref[...]
    z = z_ref[...]
    G = NUM_GROUP
    lane = jax.lax.broadcasted_iota(jnp.int32, (B, N), 1)
    slot = jax.lax.broadcasted_iota(jnp.int32, (B, G), 1)

    idx_ref[...] = jnp.zeros((B, G), dtype=jnp.int32)
    cx_ref[...] = jnp.zeros((B, G), dtype=jnp.float32)
    cy_ref[...] = jnp.zeros((B, G), dtype=jnp.float32)
    cz_ref[...] = jnp.zeros((B, G), dtype=jnp.float32)

    def body(t, carry):
        dists, idx = carry
        onehot = lane == idx
        cx = jnp.sum(jnp.where(onehot, x, 0.0), axis=1, keepdims=True)
        cy = jnp.sum(jnp.where(onehot, y, 0.0), axis=1, keepdims=True)
        cz = jnp.sum(jnp.where(onehot, z, 0.0), axis=1, keepdims=True)
        rec = slot == t
        recf = jnp.where(rec, 1.0, 0.0)
        idx_ref[...] = idx_ref[...] + jnp.where(rec, 1, 0) * idx
        cx_ref[...] = cx_ref[...] + recf * cx
        cy_ref[...] = cy_ref[...] + recf * cy
        cz_ref[...] = cz_ref[...] + recf * cz
        d = (x - cx) ** 2 + (y - cy) ** 2 + (z - cz) ** 2
        dists = jnp.minimum(dists, d)
        m = jnp.max(dists, axis=1, keepdims=True)
        nidx = jnp.min(jnp.where(dists == m, lane, N), axis=1, keepdims=True)
        return dists, nidx

    init = (
        jnp.full((B, N), jnp.inf, dtype=jnp.float32),
        jnp.zeros((B, 1), dtype=jnp.int32),
    )
    jax.lax.fori_loop(0, NUM_GROUP, body, init)


def _fps(x, y, z):
    G = NUM_GROUP
    return pl.pallas_call(
        _fps_kernel,
        out_shape=(
            jax.ShapeDtypeStruct((B, G), jnp.int32),
            jax.ShapeDtypeStruct((B, G), jnp.float32),
            jax.ShapeDtypeStruct((B, G), jnp.float32),
            jax.ShapeDtypeStruct((B, G), jnp.float32),
        ),
    )(x, y, z)


_KW = GROUP_SIZE
_RW = (B * NUM_GROUP) // 32


def _sc_gather_body(
    xs_hbm, ys_hbm, zs_hbm, idx_hbm, cx_hbm, cy_hbm, cz_hbm,
    ox_hbm, oy_hbm, oz_hbm,
    idx_v, gx_v, gy_v, gz_v, cx_v, cy_v, cz_v, ox_v, oy_v, oz_v, sem,
):
    wid = lax.axis_index("s") * 2 + lax.axis_index("c")
    pltpu.sync_copy(idx_hbm.at[wid], idx_v)
    pltpu.sync_copy(cx_hbm.at[wid], cx_v)
    pltpu.sync_copy(cy_hbm.at[wid], cy_v)
    pltpu.sync_copy(cz_hbm.at[wid], cz_v)
    handles = []
    for k in range(_KW):
        handles.append(pltpu.async_copy(xs_hbm.at[idx_v.at[k]], gx_v.at[k], sem))
        handles.append(pltpu.async_copy(ys_hbm.at[idx_v.at[k]], gy_v.at[k], sem))
        handles.append(pltpu.async_copy(zs_hbm.at[idx_v.at[k]], gz_v.at[k], sem))
    for h in handles:
        h.wait()

    for k in range(_KW):
        def rbody(rr, c, k=k):
            s = pl.ds(rr * 16, 16)
            ox_v[k, s] = gx_v[k, s] - cx_v[s]
            oy_v[k, s] = gy_v[k, s] - cy_v[s]
            oz_v[k, s] = gz_v[k, s] - cz_v[s]
            return c

        lax.fori_loop(0, _RW // 16, rbody, 0)
    pltpu.sync_copy(ox_v, ox_hbm.at[wid])
    pltpu.sync_copy(oy_v, oy_hbm.at[wid])
    pltpu.sync_copy(oz_v, oz_hbm.at[wid])


def _sc_gather(xs, ys, zs, idx_g, cx_t, cy_t, cz_t):
    mesh = plsc.VectorSubcoreMesh(core_axis_name="c", subcore_axis_name="s")
    o = jax.ShapeDtypeStruct((32, _KW, _RW), jnp.float32)
    fn = pl.kernel(
        _sc_gather_body,
        mesh=mesh,
        out_type=(o, o, o),
        scratch_types=[
            pltpu.VMEM((_KW, _RW), jnp.int32),
            pltpu.VMEM((_KW, _RW), jnp.float32),
            pltpu.VMEM((_KW, _RW), jnp.float32),
            pltpu.VMEM((_KW, _RW), jnp.float32),
            pltpu.VMEM((_RW,), jnp.float32),
            pltpu.VMEM((_RW,), jnp.float32),
            pltpu.VMEM((_RW,), jnp.float32),
            pltpu.VMEM((_KW, _RW), jnp.float32),
            pltpu.VMEM((_KW, _RW), jnp.float32),
            pltpu.VMEM((_KW, _RW), jnp.float32),
            pltpu.SemaphoreType.DMA,
        ],
    )
    return fn(xs, ys, zs, idx_g, cx_t, cy_t, cz_t)


def _gather_sub(x, y, z, cx, cy, cz, idx):
    idx_g = idx.astype(jnp.int32) + (jnp.arange(B, dtype=jnp.int32) * N)[:, None, None]
    idx_t = (
        idx_g.reshape(B, 4, _RW, _KW).transpose(0, 1, 3, 2).reshape(32, _KW, _RW)
    )
    ox, oy, oz = _sc_gather(
        x.reshape(-1), y.reshape(-1), z.reshape(-1),
        idx_t,
        cx.reshape(32, _RW), cy.reshape(32, _RW), cz.reshape(32, _RW),
    )
    nb = jnp.stack([ox, oy, oz], axis=-1)
    nb = nb.reshape(B, 4, _KW, _RW, 3).transpose(0, 1, 3, 2, 4)
    return nb.reshape(B, NUM_GROUP, GROUP_SIZE, 3)


def _dist_body(x_ref, y_ref, z_ref, cx_ref, cy_ref, cz_ref, d_ref, m1_ref, m2_ref):
    x = x_ref[0]
    y = y_ref[0]
    z = z_ref[0]
    cx = cx_ref[0, 0]
    cy = cy_ref[0, 0]
    cz = cz_ref[0, 0]
    d = (cx - x) ** 2 + (cy - y) ** 2 + (cz - z) ** 2
    d_ref[0, 0] = d
    m1 = d[:, 0:1024]
    for k in range(1, 8):
        m1 = jnp.minimum(m1, d[:, k * 1024:(k + 1) * 1024])
    m1_ref[0, 0] = m1
    m2 = m1[:, 0:128]
    for k in range(1, 8):
        m2 = jnp.minimum(m2, m1[:, k * 128:(k + 1) * 128])
    m2_ref[0, 0] = m2


def _dist(x, y, z, cxc, cyc, czc):
    return pl.pallas_call(
        _dist_body,
        grid=(B, 4),
        in_specs=[
            pl.BlockSpec((1, 1, N), lambda b, q: (b, 0, 0)),
            pl.BlockSpec((1, 1, N), lambda b, q: (b, 0, 0)),
            pl.BlockSpec((1, 1, N), lambda b, q: (b, 0, 0)),
            pl.BlockSpec((1, 1, 128, 1), lambda b, q: (b, q, 0, 0)),
            pl.BlockSpec((1, 1, 128, 1), lambda b, q: (b, q, 0, 0)),
            pl.BlockSpec((1, 1, 128, 1), lambda b, q: (b, q, 0, 0)),
        ],
        out_specs=[
            pl.BlockSpec((1, 1, 128, N), lambda b, q: (b, q, 0, 0)),
            pl.BlockSpec((1, 1, 128, 1024), lambda b, q: (b, q, 0, 0)),
            pl.BlockSpec((1, 1, 128, 128), lambda b, q: (b, q, 0, 0)),
        ],
        out_shape=[
            jax.ShapeDtypeStruct((B, 4, 128, N), jnp.float32),
            jax.ShapeDtypeStruct((B, 4, 128, 1024), jnp.float32),
            jax.ShapeDtypeStruct((B, 4, 128, 128), jnp.float32),
        ],
    )(x.reshape(B, 1, N), y.reshape(B, 1, N), z.reshape(B, 1, N), cxc, cyc, czc)


def _topk_body(vals_ref, sel_ref, out_ref, *, mode):
    v = vals_ref[0]
    M = v.shape[0]
    if mode == "slot":
        ids = jax.lax.broadcasted_iota(jnp.int32, (M, 128), 0)
    else:
        stride = 128 if mode == "m1" else 1024
        sel = sel_ref[0]
        rep = jnp.concatenate([sel] * (M // 32), axis=0)
        kpat = (jax.lax.broadcasted_iota(jnp.int32, (M, 128), 0) // 32) * stride
        ids = rep + kpat
    bigi = jnp.int32(1 << 30)
    inf = jnp.float32(jnp.inf)
    for t in range(GROUP_SIZE):
        m = jnp.min(v, axis=0, keepdims=True)
        eq = v == m
        sid = jnp.min(jnp.where(eq, ids, bigi), axis=0, keepdims=True)
        out_ref[0, t] = sid[0]
        v = jnp.where(eq & (ids == sid), inf, v)


def _topk(vals, sel, mode, grid):
    import functools as _ft

    M = vals.shape[-2]
    nsel = vals.shape[0] * (1 if vals.ndim == 3 else vals.shape[1])
    if vals.ndim == 3:
        vspec = pl.BlockSpec((1, M, 128), lambda p: (p, 0, 0))
        sspec = pl.BlockSpec((1, 32, 128), lambda p: (p, 0, 0))
        ospec = pl.BlockSpec((1, 32, 128), lambda p: (p, 0, 0))
        oshape = jax.ShapeDtypeStruct((vals.shape[0], 32, 128), jnp.int32)
    else:
        raise ValueError
    args = [vals] if mode == "slot" else [vals, sel]
    in_specs = [vspec] if mode == "slot" else [vspec, sspec]

    def body(*refs):
        if mode == "slot":
            _topk_body(refs[0], None, refs[1], mode=mode)
        else:
            _topk_body(refs[0], refs[1], refs[2], mode=mode)

    return pl.pallas_call(
        body,
        grid=grid,
        in_specs=in_specs,
        out_specs=ospec,
        out_shape=oshape,
    )(*args)


def _sc_selgather_body(row_words, k_stride, tab_hbm, sel_hbm, out_hbm,
                       sel_v, idx_v, out_v, sem):
    wid = lax.axis_index("s") * 2 + lax.axis_index("c")
    pltpu.sync_copy(sel_hbm.at[wid], sel_v)
    for chunk in range(8):
        def fire(i, c, chunk=chunk):
            rr = chunk * 16 + i
            base = (wid * 128 + rr) * row_words
            s0 = sel_v[pl.ds(rr * 32, 16)] + base
            s1 = sel_v[pl.ds(rr * 32 + 16, 16)] + base
            for k in range(8):
                h = k // 4
                off = i * 256 + h * 128 + (k % 4) * 32
                idx_v[pl.ds(off, 16)] = s0 + (k * k_stride)
                idx_v[pl.ds(off + 16, 16)] = s1 + (k * k_stride)
            for h in range(2):
                pltpu.async_copy(
                    tab_hbm.at[idx_v.at[pl.ds(i * 256 + h * 128, 128)]],
                    out_v.at[rr, h],
                    sem,
                )
            return c

        lax.fori_loop(0, 16, fire, 0)
        for i in range(16):
            for h in range(2):
                pltpu.make_async_copy(
                    tab_hbm.at[pl.ds(0, 128)],
                    out_v.at[chunk * 16 + i, h],
                    sem,
                ).wait()
    pltpu.sync_copy(out_v, out_hbm.at[wid])


def _sc_selgather(tab, sel_t, row_words, k_stride):
    import functools as _ft

    mesh = plsc.VectorSubcoreMesh(core_axis_name="c", subcore_axis_name="s")
    fn = pl.kernel(
        _ft.partial(_sc_selgather_body, row_words, k_stride),
        mesh=mesh,
        out_type=jax.ShapeDtypeStruct((32, 128, 2, 128), jnp.float32),
        scratch_types=[
            pltpu.VMEM((4096,), jnp.int32),
            pltpu.VMEM((4096,), jnp.int32),
            pltpu.VMEM((128, 2, 128), jnp.float32),
            pltpu.SemaphoreType.DMA,
        ],
    )
    return fn(tab, sel_t)


def kernel(xyz):
    x = xyz[:, :, 0]
    y = xyz[:, :, 1]
    z = xyz[:, :, 2]
    fps_idx, cx, cy, cz = _fps(x, y, z)
    center = jnp.stack([cx, cy, cz], axis=-1)

    cxc = cx.reshape(B, 4, 128, 1)
    cyc = cy.reshape(B, 4, 128, 1)
    czc = cz.reshape(B, 4, 128, 1)
    d4, m14, m24 = _dist(x, y, z, cxc, cyc, czc)
    m2t = m24.reshape(32, 128, 128).transpose(0, 2, 1)
    sel2 = _topk(m2t, None, "slot", (32,))
    sel2_sc = sel2.transpose(0, 2, 1).reshape(32, 4096)
    m1c = _sc_selgather(m14.reshape(-1), sel2_sc, 1024, 128)
    m1ct = m1c.reshape(32, 128, 256).transpose(0, 2, 1)
    sel1 = _topk(m1ct, sel2, "m1", (32,))
    sel1_sc = sel1.transpose(0, 2, 1).reshape(32, 4096)
    dc = _sc_selgather(d4.reshape(-1), sel1_sc, N, 1024)
    dct = dc.reshape(32, 128, 256).transpose(0, 2, 1)
    fidx = _topk(dct, sel1, "d", (32,))
    idx_t = fidx + (jnp.arange(32, dtype=jnp.int32) // 4 * N)[:, None, None]
    ox, oy, oz = _sc_gather(
        x.reshape(-1), y.reshape(-1), z.reshape(-1),
        idx_t,
        cx.reshape(32, _RW), cy.reshape(32, _RW), cz.reshape(32, _RW),
    )
    nb = jnp.stack([ox, oy, oz], axis=-1)
    nb = nb.reshape(B, 4, _KW, _RW, 3).transpose(0, 1, 3, 2, 4)
    neighborhood = nb.reshape(B, NUM_GROUP, GROUP_SIZE, 3)
    return (neighborhood, center)
